# Initial kernel scaffold; baseline (speedup 1.0000x reference)
import jax
import jax.numpy as jnp
from jax.experimental import pallas as pl


def kernel(z_e, codebook):
    raise NotImplementedError("write your pallas kernel here")


# speedup vs baseline: 1.5781x; 1.5781x over previous
import functools

import jax
import jax.numpy as jnp
from jax.experimental import pallas as pl

_N, _D, _K = 65536, 64, 1024
_BLOCK = 1024
_NB = _N // _BLOCK


def _vq_body(z_ref, c_ref, zq_ref, idx_ref, loss_ref):
    i = pl.program_id(0)
    z = z_ref[...]
    c = c_ref[...]
    m = jax.lax.dot_general(
        z, c, dimension_numbers=(((1,), (1,)), ((), ())),
        preferred_element_type=jnp.float32)
    a = jnp.sum(z * z, axis=1, keepdims=True)
    c2 = jnp.sum(c * c, axis=1)[None, :]
    d = (a - 2.0 * m) + c2
    idx = jnp.argmin(d, axis=1).astype(jnp.int32)
    idx_ref[0, 0, :] = idx
    onehot = (jax.lax.broadcasted_iota(jnp.int32, (_BLOCK, _K), 1)
              == idx[:, None]).astype(jnp.float32)
    zq = jax.lax.dot_general(
        onehot, c, dimension_numbers=(((1,), (0,)), ((), ())),
        preferred_element_type=jnp.float32)
    zq_ref[...] = z + (zq - z)
    diff = z - zq
    part = jnp.sum(diff * diff).reshape(1, 1)

    @pl.when(i == 0)
    def _init():
        loss_ref[...] = jnp.zeros((1, 1), jnp.float32)

    loss_ref[...] += part


@functools.partial(jax.jit)
def kernel(z_e, codebook):
    zq, idx3, loss = pl.pallas_call(
        _vq_body,
        grid=(_NB,),
        in_specs=[
            pl.BlockSpec((_BLOCK, _D), lambda i: (i, 0)),
            pl.BlockSpec((_K, _D), lambda i: (0, 0)),
        ],
        out_specs=[
            pl.BlockSpec((_BLOCK, _D), lambda i: (i, 0)),
            pl.BlockSpec((1, 1, _BLOCK), lambda i: (i, 0, 0)),
            pl.BlockSpec((1, 1), lambda i: (0, 0)),
        ],
        out_shape=[
            jax.ShapeDtypeStruct((_N, _D), jnp.float32),
            jax.ShapeDtypeStruct((_NB, 1, _BLOCK), jnp.int32),
            jax.ShapeDtypeStruct((1, 1), jnp.float32),
        ],
    )(z_e, codebook)
    indices = idx3.reshape(_N)
    commitment_loss = (loss[0, 0] / jnp.float32(_N * _D)).reshape(())
    return zq, indices, commitment_loss


# speedup vs baseline: 2.0667x; 1.3096x over previous
import functools

import jax
import jax.numpy as jnp
from jax import lax
from jax.experimental import pallas as pl
from jax.experimental.pallas import tpu as pltpu, tpu_sc as plsc

_N, _D, _K = 65536, 64, 1024
_BLOCK = 1024
_NB = _N // _BLOCK

_NC, _NS = 2, 16
_NW = _NC * _NS
_BPW = _N // _NW
_CHUNK = 128
_NCHUNK = _BPW // _CHUNK


def _dist_body(zt_ref, c_ref, idx_ref, loss_ref):
    i = pl.program_id(0)
    zt = zt_ref[...]
    c = c_ref[...]
    mT2 = jax.lax.dot_general(
        c * (-2.0), zt, dimension_numbers=(((1,), (0,)), ((), ())),
        preferred_element_type=jnp.float32)
    a = jnp.sum(zt * zt, axis=0)[None, :]
    c2 = jnp.sum(c * c, axis=1)[:, None]
    dT = (a + mT2) + c2
    t = dT
    r = _K
    while r > 8:
        h = r // 2
        t = jnp.minimum(t[:h], t[h:])
        r = h
    dmin = jnp.min(t, axis=0)
    code_ids = jax.lax.broadcasted_iota(jnp.int32, (_K, _BLOCK), 0)
    cand = jnp.where(dT == dmin[None, :], code_ids, jnp.int32(0x7FFFFFFF))
    r = _K
    while r > 8:
        h = r // 2
        cand = jnp.minimum(cand[:h], cand[h:])
        r = h
    idx_ref[0, 0, :] = jnp.min(cand, axis=0)
    part = jnp.sum(dmin).reshape(1, 1)

    @pl.when(i == 0)
    def _init():
        loss_ref[...] = jnp.zeros((1, 1), jnp.float32)

    loss_ref[...] += part


def _tc_assign(zt, codebook):
    idx3, loss = pl.pallas_call(
        _dist_body,
        grid=(_NB,),
        in_specs=[
            pl.BlockSpec((_D, _BLOCK), lambda i: (0, i)),
            pl.BlockSpec((_K, _D), lambda i: (0, 0)),
        ],
        out_specs=[
            pl.BlockSpec((1, 1, _BLOCK), lambda i: (i, 0, 0)),
            pl.BlockSpec((1, 1), lambda i: (0, 0)),
        ],
        out_shape=[
            jax.ShapeDtypeStruct((_NB, 1, _BLOCK), jnp.int32),
            jax.ShapeDtypeStruct((1, 1), jnp.float32),
        ],
    )(zt, codebook)
    return idx3.reshape(_N), loss


@functools.cache
def _make_sc_gather():
    @functools.partial(
        pl.kernel,
        mesh=plsc.VectorSubcoreMesh(core_axis_name="c", subcore_axis_name="s"),
        out_type=jax.ShapeDtypeStruct((_N, _D), jnp.float32),
        scratch_types=[
            pltpu.VMEM((_BPW,), jnp.int32),
            pltpu.VMEM((_CHUNK, _D), jnp.float32),
            pltpu.VMEM((_CHUNK, _D), jnp.float32),
            pltpu.SemaphoreType.DMA,
            pltpu.SemaphoreType.DMA,
            pltpu.SemaphoreType.DMA,
            pltpu.SemaphoreType.DMA,
        ],
        compiler_params=pltpu.CompilerParams(use_tc_tiling_on_sc=False),
    )
    def _sc_gather(table_hbm, idx_hbm, out_hbm, idx_v, rows0, rows1, g0, g1, s0, s1):
        wid = lax.axis_index("s") * _NC + lax.axis_index("c")
        base = wid * _BPW
        pltpu.sync_copy(idx_hbm.at[pl.ds(base, _BPW)], idx_v)
        rows = (rows0, rows1)
        gsem = (g0, g1)
        ssem = (s0, s1)
        scatters = [None, None]
        for chunk in range(_NCHUNK):
            b = chunk % 2
            if scatters[b] is not None:
                scatters[b].wait()
            gat = pltpu.async_copy(
                table_hbm.at[idx_v.at[pl.ds(chunk * _CHUNK, _CHUNK)]], rows[b], gsem[b])
            gat.wait()
            scatters[b] = pltpu.async_copy(
                rows[b], out_hbm.at[pl.ds(base + chunk * _CHUNK, _CHUNK)], ssem[b])
        scatters[0].wait()
        scatters[1].wait()

    return _sc_gather


def kernel(z_e, codebook):
    indices, loss = _tc_assign(z_e.T, codebook)
    z_q = _make_sc_gather()(codebook, indices)
    commitment_loss = (loss[0, 0] / jnp.float32(_N * _D)).reshape(())
    return z_q, indices, commitment_loss
